# baseline (speedup 1.0000x reference)
import functools

import jax
import jax.numpy as jnp
from jax import lax
from jax.experimental import pallas as pl
from jax.experimental.pallas import tpu as pltpu
from jax.experimental.pallas import tpu_sc as plsc

BATCH = 16384
EMB = 128
HID = 128
NOUT = 4

NUM_CORES = 2
NUM_SUBCORES = 16
NW = NUM_CORES * NUM_SUBCORES
B_PER_W = BATCH // NW
CHUNK = 128
N_CHUNKS = B_PER_W // CHUNK

BLK = 16384


def _sc_gather(table, idx2d):
    mesh = plsc.VectorSubcoreMesh(core_axis_name="c", subcore_axis_name="s")

    @functools.partial(
        pl.kernel,
        mesh=mesh,
        out_type=jax.ShapeDtypeStruct((BATCH, EMB), jnp.float32),
        scratch_types=[
            pltpu.VMEM((N_CHUNKS, CHUNK), jnp.int32),
            pltpu.VMEM((B_PER_W, EMB), jnp.float32),
        ]
        + [pltpu.SemaphoreType.DMA] * N_CHUNKS
        + [pltpu.SemaphoreType.DMA],
    )
    def gather_kernel(table_hbm, idx_hbm, out_hbm, idx_v, rows_v, *sems):
        gsems, ssem = sems[:N_CHUNKS], sems[N_CHUNKS]
        wid = lax.axis_index("s") * NUM_CORES + lax.axis_index("c")
        pltpu.sync_copy(idx_hbm.at[pl.ds(wid * N_CHUNKS, N_CHUNKS)], idx_v)
        gathers = []
        for j in range(N_CHUNKS):
            gathers.append(
                pltpu.async_copy(
                    table_hbm.at[idx_v.at[j]],
                    rows_v.at[pl.ds(j * CHUNK, CHUNK)],
                    gsems[j],
                )
            )
        stores = []
        for j in range(N_CHUNKS):
            gathers[j].wait()
            stores.append(
                pltpu.async_copy(
                    rows_v.at[pl.ds(j * CHUNK, CHUNK)],
                    out_hbm.at[pl.ds(wid * B_PER_W + j * CHUNK, CHUNK)],
                    ssem,
                )
            )
        for s in stores:
            s.wait()

    return gather_kernel(table, idx2d)


def _mlp_body(x_ref, w1_ref, b1_ref, w2_ref, b2_ref, w3t_ref, b3t_ref, o_ref):
    x = x_ref[...]
    h = jnp.dot(x, w1_ref[...], preferred_element_type=jnp.float32)
    h = jnp.maximum(h + b1_ref[...], 0.0)
    h = jnp.dot(h, w2_ref[...], preferred_element_type=jnp.float32)
    h = jnp.maximum(h + b2_ref[...], 0.0)
    ot = lax.dot_general(
        w3t_ref[...], h, (((1,), (1,)), ((), ())),
        preferred_element_type=jnp.float32,
    )
    o_ref[...] = ot + b3t_ref[...]


def _tc_mlp(emb, W1, b1, W2, b2, W3, b3):
    w3t = W3.T
    b3t = jnp.broadcast_to(b3.reshape(NOUT, 1), (NOUT, BLK))
    grid = (BATCH // BLK,)
    out = pl.pallas_call(
        _mlp_body,
        grid=grid,
        in_specs=[
            pl.BlockSpec((BLK, EMB), lambda i: (i, 0)),
            pl.BlockSpec((EMB, HID), lambda i: (0, 0)),
            pl.BlockSpec((1, HID), lambda i: (0, 0)),
            pl.BlockSpec((HID, HID), lambda i: (0, 0)),
            pl.BlockSpec((1, HID), lambda i: (0, 0)),
            pl.BlockSpec((NOUT, HID), lambda i: (0, 0)),
            pl.BlockSpec((NOUT, BLK), lambda i: (0, 0)),
        ],
        out_specs=pl.BlockSpec((NOUT, BLK), lambda i: (0, i)),
        out_shape=jax.ShapeDtypeStruct((NOUT, BATCH), jnp.float32),
        compiler_params=pltpu.CompilerParams(
            dimension_semantics=("parallel",),
        ),
    )(emb, W1, b1.reshape(1, HID), W2, b2.reshape(1, HID), w3t, b3t)
    return out.T


def kernel(state_idx, table, W1, b1, W2, b2, W3, b3):
    idx2d = state_idx.reshape(NW * N_CHUNKS, CHUNK)
    emb = _sc_gather(table, idx2d)
    return _tc_mlp(emb, W1, b1, W2, b2, W3, b3)


# speedup vs baseline: 1.0078x; 1.0078x over previous
import functools

import jax
import jax.numpy as jnp
from jax import lax
from jax.experimental import pallas as pl
from jax.experimental.pallas import tpu as pltpu
from jax.experimental.pallas import tpu_sc as plsc

BATCH = 16384
EMB = 128
HID = 128
NOUT = 4

NUM_CORES = 2
NUM_SUBCORES = 16
NW = NUM_CORES * NUM_SUBCORES
B_PER_W = BATCH // NW
CHUNK = 64
N_CHUNKS = B_PER_W // CHUNK

BLK = 8192


def _sc_gather(table, idx2d):
    mesh = plsc.VectorSubcoreMesh(core_axis_name="c", subcore_axis_name="s")

    @functools.partial(
        pl.kernel,
        mesh=mesh,
        out_type=jax.ShapeDtypeStruct((BATCH, EMB), jnp.float32),
        scratch_types=[
            pltpu.VMEM((N_CHUNKS, CHUNK), jnp.int32),
            pltpu.VMEM((B_PER_W, EMB), jnp.float32),
        ]
        + [pltpu.SemaphoreType.DMA] * N_CHUNKS
        + [pltpu.SemaphoreType.DMA],
    )
    def gather_kernel(table_hbm, idx_hbm, out_hbm, idx_v, rows_v, *sems):
        gsems, ssem = sems[:N_CHUNKS], sems[N_CHUNKS]
        wid = lax.axis_index("s") * NUM_CORES + lax.axis_index("c")
        pltpu.sync_copy(idx_hbm.at[pl.ds(wid * N_CHUNKS, N_CHUNKS)], idx_v)
        gathers = []
        for j in range(N_CHUNKS):
            gathers.append(
                pltpu.async_copy(
                    table_hbm.at[idx_v.at[j]],
                    rows_v.at[pl.ds(j * CHUNK, CHUNK)],
                    gsems[j],
                )
            )
        stores = []
        for j in range(N_CHUNKS):
            gathers[j].wait()
            stores.append(
                pltpu.async_copy(
                    rows_v.at[pl.ds(j * CHUNK, CHUNK)],
                    out_hbm.at[pl.ds(wid * B_PER_W + j * CHUNK, CHUNK)],
                    ssem,
                )
            )
        for s in stores:
            s.wait()

    return gather_kernel(table, idx2d)


def _mlp_body(x_ref, w1_ref, b1_ref, w2_ref, b2_ref, w3t_ref, b3t_ref, o_ref):
    x = x_ref[...]
    h = jnp.dot(x, w1_ref[...], preferred_element_type=jnp.float32)
    h = jnp.maximum(h + b1_ref[...], 0.0)
    h = jnp.dot(h, w2_ref[...], preferred_element_type=jnp.float32)
    h = jnp.maximum(h + b2_ref[...], 0.0)
    ot = lax.dot_general(
        w3t_ref[...], h, (((1,), (1,)), ((), ())),
        preferred_element_type=jnp.float32,
    )
    o_ref[...] = ot + b3t_ref[...]


def _tc_mlp(emb, W1, b1, W2, b2, W3, b3):
    w3t = W3.T
    b3t = jnp.broadcast_to(b3.reshape(NOUT, 1), (NOUT, BLK))
    grid = (BATCH // BLK,)
    out = pl.pallas_call(
        _mlp_body,
        grid=grid,
        in_specs=[
            pl.BlockSpec((BLK, EMB), lambda i: (i, 0)),
            pl.BlockSpec((EMB, HID), lambda i: (0, 0)),
            pl.BlockSpec((1, HID), lambda i: (0, 0)),
            pl.BlockSpec((HID, HID), lambda i: (0, 0)),
            pl.BlockSpec((1, HID), lambda i: (0, 0)),
            pl.BlockSpec((NOUT, HID), lambda i: (0, 0)),
            pl.BlockSpec((NOUT, BLK), lambda i: (0, 0)),
        ],
        out_specs=pl.BlockSpec((NOUT, BLK), lambda i: (0, i)),
        out_shape=jax.ShapeDtypeStruct((NOUT, BATCH), jnp.float32),
        compiler_params=pltpu.CompilerParams(
            dimension_semantics=("parallel",),
        ),
    )(emb, W1, b1.reshape(1, HID), W2, b2.reshape(1, HID), w3t, b3t)
    return out.T


def kernel(state_idx, table, W1, b1, W2, b2, W3, b3):
    idx2d = state_idx.reshape(NW * N_CHUNKS, CHUNK)
    emb = _sc_gather(table, idx2d)
    return _tc_mlp(emb, W1, b1, W2, b2, W3, b3)


# speedup vs baseline: 1.0225x; 1.0146x over previous
import functools

import jax
import jax.numpy as jnp
from jax import lax
from jax.experimental import pallas as pl
from jax.experimental.pallas import tpu as pltpu
from jax.experimental.pallas import tpu_sc as plsc

BATCH = 16384
EMB = 128
HID = 128
NOUT = 4

NUM_CORES = 2
NUM_SUBCORES = 16
NW = NUM_CORES * NUM_SUBCORES
B_PER_W = BATCH // NW
CHUNK = 128
N_CHUNKS = B_PER_W // CHUNK

BLK = 8192


def _sc_gather(table, idx2d):
    mesh = plsc.VectorSubcoreMesh(core_axis_name="c", subcore_axis_name="s")

    @functools.partial(
        pl.kernel,
        mesh=mesh,
        out_type=jax.ShapeDtypeStruct((BATCH, EMB), jnp.float32),
        scratch_types=[
            pltpu.VMEM((N_CHUNKS, CHUNK), jnp.int32),
            pltpu.VMEM((B_PER_W, EMB), jnp.float32),
        ]
        + [pltpu.SemaphoreType.DMA] * N_CHUNKS
        + [pltpu.SemaphoreType.DMA],
    )
    def gather_kernel(table_hbm, idx_hbm, out_hbm, idx_v, rows_v, *sems):
        gsems, ssem = sems[:N_CHUNKS], sems[N_CHUNKS]
        wid = lax.axis_index("s") * NUM_CORES + lax.axis_index("c")
        pltpu.sync_copy(idx_hbm.at[pl.ds(wid * N_CHUNKS, N_CHUNKS)], idx_v)
        gathers = []
        for j in range(N_CHUNKS):
            gathers.append(
                pltpu.async_copy(
                    table_hbm.at[idx_v.at[j]],
                    rows_v.at[pl.ds(j * CHUNK, CHUNK)],
                    gsems[j],
                )
            )
        stores = []
        for j in range(N_CHUNKS):
            gathers[j].wait()
            stores.append(
                pltpu.async_copy(
                    rows_v.at[pl.ds(j * CHUNK, CHUNK)],
                    out_hbm.at[pl.ds(wid * B_PER_W + j * CHUNK, CHUNK)],
                    ssem,
                )
            )
        for s in stores:
            s.wait()

    return gather_kernel(table, idx2d)


def _mlp_body(x_ref, w1_ref, b1_ref, w2_ref, b2_ref, w3t_ref, b3t_ref, o_ref):
    x = x_ref[...]
    h = jnp.dot(x, w1_ref[...], preferred_element_type=jnp.float32)
    h = jnp.maximum(h + b1_ref[...], 0.0)
    h = jnp.dot(h, w2_ref[...], preferred_element_type=jnp.float32)
    h = jnp.maximum(h + b2_ref[...], 0.0)
    ot = lax.dot_general(
        w3t_ref[...], h, (((1,), (1,)), ((), ())),
        preferred_element_type=jnp.float32,
    )
    o_ref[...] = ot + b3t_ref[...]


def _tc_mlp(emb, W1, b1, W2, b2, W3, b3):
    w3t = W3.T
    b3t = jnp.broadcast_to(b3.reshape(NOUT, 1), (NOUT, BLK))
    grid = (BATCH // BLK,)
    out = pl.pallas_call(
        _mlp_body,
        grid=grid,
        in_specs=[
            pl.BlockSpec((BLK, EMB), lambda i: (i, 0)),
            pl.BlockSpec((EMB, HID), lambda i: (0, 0)),
            pl.BlockSpec((1, HID), lambda i: (0, 0)),
            pl.BlockSpec((HID, HID), lambda i: (0, 0)),
            pl.BlockSpec((1, HID), lambda i: (0, 0)),
            pl.BlockSpec((NOUT, HID), lambda i: (0, 0)),
            pl.BlockSpec((NOUT, BLK), lambda i: (0, 0)),
        ],
        out_specs=pl.BlockSpec((NOUT, BLK), lambda i: (0, i)),
        out_shape=jax.ShapeDtypeStruct((NOUT, BATCH), jnp.float32),
        compiler_params=pltpu.CompilerParams(
            dimension_semantics=("parallel",),
        ),
    )(emb, W1, b1.reshape(1, HID), W2, b2.reshape(1, HID), w3t, b3t)
    return out.T


def kernel(state_idx, table, W1, b1, W2, b2, W3, b3):
    idx2d = state_idx.reshape(NW * N_CHUNKS, CHUNK)
    emb = _sc_gather(table, idx2d)
    return _tc_mlp(emb, W1, b1, W2, b2, W3, b3)
